# baseline (speedup 1.0000x reference)
import functools

import jax
import jax.numpy as jnp
from jax import lax
from jax.experimental import pallas as pl
from jax.experimental.pallas import tpu as pltpu
from jax.experimental.pallas import tpu_sc as plsc

N = 10000
E = 320000
D = 128
H = 128

NPAD = 10240
BLK = 512
GRID = NPAD // BLK

NW = 32
CHUNK = 80
CPW = 128
EPW = CPW * CHUNK
EPAD = NW * EPW
RPW = NPAD // 16
UNROLL = 4
GSPLIT = 5


def _prologue_body(x_ref, ws_ref, wd_ref, atts_ref, attd_ref,
                   xs_ref, asad_ref, mx_ref):
    i = pl.program_id(0)
    xb = x_ref[...]
    xs = jnp.dot(xb, ws_ref[...], preferred_element_type=jnp.float32)
    xs_ref[...] = xs.astype(jnp.bfloat16)
    a_s = jnp.dot(xs, atts_ref[...], preferred_element_type=jnp.float32)
    wdat = jnp.dot(wd_ref[...], attd_ref[...],
                   preferred_element_type=jnp.float32)
    a_d = jnp.dot(xb, wdat, preferred_element_type=jnp.float32)
    asad_ref[...] = jnp.concatenate([a_s, a_d], axis=1)
    bm = jnp.max(a_s)

    @pl.when(i == 0)
    def _():
        for j in range(16):
            mx_ref[j] = bm

    @pl.when(i > 0)
    def _():
        for j in range(16):
            mx_ref[j] = jnp.maximum(mx_ref[j], bm)


def _prologue(xp, Ws, Wd, atts, attd):
    return pl.pallas_call(
        _prologue_body,
        grid=(GRID,),
        in_specs=[
            pl.BlockSpec((BLK, D), lambda i: (i, 0)),
            pl.BlockSpec((D, H), lambda i: (0, 0)),
            pl.BlockSpec((D, H), lambda i: (0, 0)),
            pl.BlockSpec((H, 1), lambda i: (0, 0)),
            pl.BlockSpec((H, 1), lambda i: (0, 0)),
        ],
        out_specs=[
            pl.BlockSpec((BLK, H), lambda i: (i, 0)),
            pl.BlockSpec((BLK, 2), lambda i: (i, 0)),
            pl.BlockSpec(memory_space=pltpu.SMEM),
        ],
        out_shape=[
            jax.ShapeDtypeStruct((NPAD, H), jnp.bfloat16),
            jax.ShapeDtypeStruct((NPAD, 2), jnp.float32),
            jax.ShapeDtypeStruct((16,), jnp.float32),
        ],
    )(xp, Ws, Wd, atts.reshape(H, 1), attd.reshape(H, 1))


def _epilogue_body(acc_ref, s_ref, x_ref, wl_ref, b_ref, bl_ref, o_ref,
                   *, relu):
    acc = acc_ref[0] + acc_ref[1]
    s = jnp.sum(s_ref[...], axis=0)
    gat = acc / (s[:, None] + 1e-16) + b_ref[...]
    lin = jnp.dot(x_ref[...], wl_ref[...],
                  preferred_element_type=jnp.float32) + bl_ref[...]
    r = gat + lin
    o_ref[...] = jnp.maximum(r, 0.0) if relu else r


def _epilogue(acc, svals, xp, Wl, b, bl, relu):
    return pl.pallas_call(
        functools.partial(_epilogue_body, relu=relu),
        grid=(GRID,),
        in_specs=[
            pl.BlockSpec((2, BLK, H), lambda i: (0, i, 0)),
            pl.BlockSpec((2, BLK), lambda i: (0, i)),
            pl.BlockSpec((BLK, D), lambda i: (i, 0)),
            pl.BlockSpec((D, H), lambda i: (0, 0)),
            pl.BlockSpec((1, H), lambda i: (0, 0)),
            pl.BlockSpec((1, H), lambda i: (0, 0)),
        ],
        out_specs=pl.BlockSpec((BLK, H), lambda i: (i, 0)),
        out_shape=jax.ShapeDtypeStruct((NPAD, H), jnp.float32),
    )(acc, svals, xp, Wl, b.reshape(1, H), bl.reshape(1, H))


def _sc_edge_body(src_hbm, dst_hbm, asad_hbm, mx_hbm, xs_hbm,
                  zrows_hbm, zvec_hbm, acc_out, s_out,
                  asad_v, src_c, dst_c, val_c, rows_v, rows_f, mx_v,
                  si0, si1, si2, si3, sg0, sg1, sc0, ss0, ss1,
                  acc_sh, s_sh):
    cid = lax.axis_index("c")
    sid = lax.axis_index("s")
    wid = sid * 2 + cid
    sem_i = (si0, si1, si2, si3)
    sem_g = (sg0, sg1)
    sem_sc = sc0
    sem_ss = (ss0, ss1)

    pltpu.sync_copy(asad_hbm, asad_v)
    pltpu.sync_copy(mx_hbm, mx_v)
    pltpu.sync_copy(zrows_hbm.at[pl.ds(sid * RPW, RPW)],
                    acc_sh.at[pl.ds(sid * RPW, RPW)])

    @pl.when(sid == 0)
    def _():
        pltpu.sync_copy(zvec_hbm, s_sh)

    plsc.subcore_barrier()

    mxa = mx_v[...]

    def idx_start(cc, uu):
        pltpu.async_copy(src_hbm.at[wid].at[cc], src_c.at[uu], sem_i[uu])
        pltpu.async_copy(dst_hbm.at[wid].at[cc], dst_c.at[uu], sem_i[uu])

    def idx_wait(cc, uu):
        pltpu.make_async_copy(src_hbm.at[wid].at[cc], src_c.at[uu],
                              sem_i[uu]).wait()
        pltpu.make_async_copy(dst_hbm.at[wid].at[cc], dst_c.at[uu],
                              sem_i[uu]).wait()

    GS = CHUNK // GSPLIT

    def gather_start(uu):
        for p in range(GSPLIT):
            pltpu.async_copy(
                xs_hbm.at[src_c.at[uu].at[pl.ds(p * GS, GS)]],
                rows_v.at[uu % 2].at[pl.ds(p * GS, GS)],
                sem_g[uu % 2])

    def gather_wait(uu):
        for p in range(GSPLIT):
            pltpu.make_async_copy(
                xs_hbm.at[src_c.at[uu].at[pl.ds(p * GS, GS)]],
                rows_v.at[uu % 2].at[pl.ds(p * GS, GS)],
                sem_g[uu % 2]).wait()

    def rows_scatter_start(uu):
        pltpu.async_copy(rows_f, acc_sh.at[dst_c.at[uu]], sem_sc, add=True)

    def rows_scatter_wait(uu):
        pltpu.make_async_copy(rows_f, acc_sh.at[dst_c.at[uu]],
                              sem_sc).wait()

    def s_scatter_start(uu):
        pltpu.async_copy(val_c.at[uu % 2], s_sh.at[dst_c.at[uu]],
                         sem_ss[uu % 2], add=True)

    def s_scatter_wait(uu):
        pltpu.make_async_copy(val_c.at[uu % 2], s_sh.at[dst_c.at[uu]],
                              sem_ss[uu % 2]).wait()

    def val_compute(uu):
        for i in range(CHUNK // 16):
            s_idx = src_c[uu, pl.ds(i * 16, 16)]
            d_idx = dst_c[uu, pl.ds(i * 16, 16)]
            a_s = plsc.load_gather(asad_v, [s_idx * 2])
            a_d = plsc.load_gather(asad_v, [d_idx * 2 + 1])
            t = a_s + a_d
            t = jnp.where(t >= 0.0, t, 0.2 * t)
            m = mxa + a_d
            m = jnp.where(m >= 0.0, m, 0.2 * m)
            val_c[uu % 2, pl.ds(i * 16, 16)] = jnp.exp(t - m)

    def scale_scatter(uu):
        gather_wait(uu)

        @pl.loop(0, CHUNK, unroll=2)
        def _scale(j):
            vb = plsc.load_gather(val_c.at[uu % 2],
                                  [jnp.full((16,), j, jnp.int32)])
            for k in range(4):
                w = rows_v[uu % 2, j, pl.ds(k * 16, 16)]
                ab = plsc.bitcast(w, jnp.bfloat16)
                a, b = plsc.unpack(ab, format=plsc.PackFormat.INTERLEAVED)
                rows_f[j, pl.ds(k * 32, 16)] = a * vb
                rows_f[j, pl.ds(k * 32 + 16, 16)] = b * vb

        rows_scatter_start(uu)

    idx_start(0, 0)
    idx_start(1, 1)

    @pl.loop(0, CPW // UNROLL)
    def _blk(g):
        for u in range(UNROLL):
            c = g * UNROLL + u
            um1 = (u - 1) % UNROLL
            up2 = (u + 2) % UNROLL

            def drains():
                rows_scatter_wait(up2)
                s_scatter_wait(up2)

            if u < 2:
                @pl.when(g > 0)
                def _():
                    drains()
            else:
                drains()
            idx_wait(c, u)
            gather_start(u)
            if u < 2:
                idx_start(c + 2, up2)
            else:
                @pl.when(g < CPW // UNROLL - 1)
                def _():
                    idx_start(c + 2, up2)
            val_compute(u)
            s_scatter_start(u)
            if u == 0:
                @pl.when(g > 0)
                def _():
                    scale_scatter(3)
            else:
                scale_scatter(u - 1)

    rows_scatter_wait(2)
    scale_scatter(3)
    rows_scatter_wait(3)
    s_scatter_wait(2)
    s_scatter_wait(3)

    plsc.subcore_barrier()

    pltpu.sync_copy(acc_sh.at[pl.ds(sid * RPW, RPW)],
                    acc_out.at[cid].at[pl.ds(sid * RPW, RPW)])

    @pl.when(sid == 0)
    def _():
        pltpu.sync_copy(s_sh, s_out.at[cid])


def _sc_edge(src3, dst3, asad_flat, mx, xs, zrows, zvec):
    mesh = plsc.VectorSubcoreMesh(core_axis_name="c", subcore_axis_name="s")
    f = pl.kernel(
        _sc_edge_body,
        out_type=[
            jax.ShapeDtypeStruct((2, NPAD, H), jnp.float32),
            jax.ShapeDtypeStruct((2, NPAD), jnp.float32),
        ],
        mesh=mesh,
        compiler_params=pltpu.CompilerParams(needs_layout_passes=False,
                                             use_tc_tiling_on_sc=False),
        scratch_types=[
            pltpu.VMEM((2 * NPAD,), jnp.float32),
            pltpu.VMEM((UNROLL, CHUNK), jnp.int32),
            pltpu.VMEM((UNROLL, CHUNK), jnp.int32),
            pltpu.VMEM((2, CHUNK), jnp.float32),
            pltpu.VMEM((2, CHUNK, H // 2), jnp.int32),
            pltpu.VMEM((CHUNK, H), jnp.float32),
            pltpu.VMEM((16,), jnp.float32),
        ] + [pltpu.SemaphoreType.DMA] * 9 + [
            pltpu.VMEM_SHARED((NPAD, H), jnp.float32),
            pltpu.VMEM_SHARED((NPAD,), jnp.float32),
        ],
    )
    return f(src3, dst3, asad_flat, mx, xs, zrows, zvec)


_PERM = []
for _k in range(H // 32):
    for _i in range(16):
        _PERM += [_k * 32 + _i, _k * 32 + 16 + _i]
_PERM = tuple(_PERM)


def kernel(x, edge_index, Ws1, Wd1, as1, ad1, b1, Wl1, bl1,
           Ws2, Wd2, as2, ad2, b2, Wl2, bl2):
    src = edge_index[0].astype(jnp.int32)
    dst = edge_index[1].astype(jnp.int32)
    pad = EPAD - E
    src3 = jnp.concatenate([src, jnp.full((pad,), N, jnp.int32)]
                           ).reshape(NW, CPW, CHUNK)
    dst3 = jnp.concatenate([dst, jnp.full((pad,), N, jnp.int32)]
                           ).reshape(NW, CPW, CHUNK)
    xp = jnp.pad(x, ((0, NPAD - N), (0, 0)))
    zrows = jnp.zeros((NPAD, H), jnp.float32)
    zvec = jnp.zeros((NPAD,), jnp.float32)
    perm = jnp.asarray(_PERM, jnp.int32)

    def layer(xin, Ws, Wd, atts, attd, b, Wl, bl, relu):
        xs, asad, mx = _prologue(xin, Ws[:, perm], Wd, atts[perm], attd)
        xsw = jax.lax.bitcast_convert_type(
            xs.reshape(NPAD, H // 2, 2), jnp.int32)
        acc, svals = _sc_edge(src3, dst3, asad.reshape(2 * NPAD), mx,
                              xsw, zrows, zvec)
        return _epilogue(acc, svals, xin, Wl, b, bl, relu)

    h = layer(xp, Ws1, Wd1, as1, ad1, b1, Wl1, bl1, True)
    out = layer(h, Ws2, Wd2, as2, ad2, b2, Wl2, bl2, False)
    return out[:N]


# speedup vs baseline: 1.0446x; 1.0446x over previous
import functools

import jax
import jax.numpy as jnp
from jax import lax
from jax.experimental import pallas as pl
from jax.experimental.pallas import tpu as pltpu
from jax.experimental.pallas import tpu_sc as plsc

N = 10000
E = 320000
D = 128
H = 128

NPAD = 10240
BLK = 512
GRID = NPAD // BLK

NW = 32
CHUNK = 80
CPW = 128
EPW = CPW * CHUNK
EPAD = NW * EPW
RPW = NPAD // 16
UNROLL = 4
GSPLIT = 5


def _dense_from(xb, ws_ref, wd_ref, atts_ref, attd_ref, wl_ref, bl_ref,
                xs_ref, asad_ref, mx_ref, lin_ref, i):
    xs = jnp.dot(xb, ws_ref[...], preferred_element_type=jnp.float32)
    xs_ref[...] = xs.astype(jnp.bfloat16)
    a_s = jnp.dot(xs, atts_ref[...], preferred_element_type=jnp.float32)
    wdat = jnp.dot(wd_ref[...], attd_ref[...],
                   preferred_element_type=jnp.float32)
    a_d = jnp.dot(xb, wdat, preferred_element_type=jnp.float32)
    asad_ref[...] = jnp.concatenate([a_s, a_d], axis=1)
    lin_ref[...] = jnp.dot(xb, wl_ref[...],
                           preferred_element_type=jnp.float32) + bl_ref[...]
    bm = jnp.max(a_s)

    @pl.when(i == 0)
    def _():
        for j in range(16):
            mx_ref[j] = bm

    @pl.when(i > 0)
    def _():
        for j in range(16):
            mx_ref[j] = jnp.maximum(mx_ref[j], bm)


_DENSE_OUT_SPECS = [
    pl.BlockSpec((BLK, H), lambda i: (i, 0)),
    pl.BlockSpec((BLK, 2), lambda i: (i, 0)),
    pl.BlockSpec(memory_space=pltpu.SMEM),
    pl.BlockSpec((BLK, H), lambda i: (i, 0)),
]
_DENSE_OUT_SHAPE = [
    jax.ShapeDtypeStruct((NPAD, H), jnp.bfloat16),
    jax.ShapeDtypeStruct((NPAD, 2), jnp.float32),
    jax.ShapeDtypeStruct((16,), jnp.float32),
    jax.ShapeDtypeStruct((NPAD, H), jnp.float32),
]
_W_SPECS = [
    pl.BlockSpec((D, H), lambda i: (0, 0)),
    pl.BlockSpec((D, H), lambda i: (0, 0)),
    pl.BlockSpec((H, 1), lambda i: (0, 0)),
    pl.BlockSpec((H, 1), lambda i: (0, 0)),
    pl.BlockSpec((D, H), lambda i: (0, 0)),
    pl.BlockSpec((1, H), lambda i: (0, 0)),
]


def _prologue_body(x_ref, ws_ref, wd_ref, atts_ref, attd_ref, wl_ref,
                   bl_ref, xs_ref, asad_ref, mx_ref, lin_ref):
    _dense_from(x_ref[...], ws_ref, wd_ref, atts_ref, attd_ref, wl_ref,
                bl_ref, xs_ref, asad_ref, mx_ref, lin_ref, pl.program_id(0))


def _prologue(xp, Ws, Wd, atts, attd, Wl, bl):
    return pl.pallas_call(
        _prologue_body,
        grid=(GRID,),
        in_specs=[pl.BlockSpec((BLK, D), lambda i: (i, 0))] + _W_SPECS,
        out_specs=_DENSE_OUT_SPECS,
        out_shape=_DENSE_OUT_SHAPE,
    )(xp, Ws, Wd, atts.reshape(H, 1), attd.reshape(H, 1), Wl,
      bl.reshape(1, H))


def _mid_body(acc_ref, s_ref, lin_ref, b_ref, ws_ref, wd_ref, atts_ref,
              attd_ref, wl_ref, bl_ref, xs_ref, asad_ref, mx_ref, lin2_ref):
    acc = acc_ref[0] + acc_ref[1]
    s = s_ref[0] + s_ref[1]
    gat = acc / (s[:, None] + 1e-16) + b_ref[...]
    h = jnp.maximum(gat + lin_ref[...], 0.0)
    _dense_from(h, ws_ref, wd_ref, atts_ref, attd_ref, wl_ref, bl_ref,
                xs_ref, asad_ref, mx_ref, lin2_ref, pl.program_id(0))


def _mid(acc, svals, lin, b, Ws, Wd, atts, attd, Wl, bl):
    return pl.pallas_call(
        _mid_body,
        grid=(GRID,),
        in_specs=[
            pl.BlockSpec((2, BLK, H), lambda i: (0, i, 0)),
            pl.BlockSpec((2, BLK), lambda i: (0, i)),
            pl.BlockSpec((BLK, H), lambda i: (i, 0)),
            pl.BlockSpec((1, H), lambda i: (0, 0)),
        ] + _W_SPECS,
        out_specs=_DENSE_OUT_SPECS,
        out_shape=_DENSE_OUT_SHAPE,
    )(acc, svals, lin, b.reshape(1, H), Ws, Wd, atts.reshape(H, 1),
      attd.reshape(H, 1), Wl, bl.reshape(1, H))


def _final_body(acc_ref, s_ref, lin_ref, b_ref, o_ref):
    acc = acc_ref[0] + acc_ref[1]
    s = s_ref[0] + s_ref[1]
    o_ref[...] = acc / (s[:, None] + 1e-16) + b_ref[...] + lin_ref[...]


def _final(acc, svals, lin, b):
    return pl.pallas_call(
        _final_body,
        grid=(GRID,),
        in_specs=[
            pl.BlockSpec((2, BLK, H), lambda i: (0, i, 0)),
            pl.BlockSpec((2, BLK), lambda i: (0, i)),
            pl.BlockSpec((BLK, H), lambda i: (i, 0)),
            pl.BlockSpec((1, H), lambda i: (0, 0)),
        ],
        out_specs=pl.BlockSpec((BLK, H), lambda i: (i, 0)),
        out_shape=jax.ShapeDtypeStruct((NPAD, H), jnp.float32),
    )(acc, svals, lin, b.reshape(1, H))


def _sc_edge_body(src_hbm, dst_hbm, asad_hbm, mx_hbm, xs_hbm,
                  zrows_hbm, zvec_hbm, acc_out, s_out,
                  asad_v, src_c, dst_c, val_c, rows_v, rows_f, mx_v,
                  si0, si1, si2, si3, sg0, sg1, sc0, ss0, ss1,
                  acc_sh, s_sh):
    cid = lax.axis_index("c")
    sid = lax.axis_index("s")
    wid = sid * 2 + cid
    sem_i = (si0, si1, si2, si3)
    sem_g = (sg0, sg1)
    sem_sc = sc0
    sem_ss = (ss0, ss1)

    pltpu.sync_copy(asad_hbm, asad_v)
    pltpu.sync_copy(mx_hbm, mx_v)
    pltpu.sync_copy(zrows_hbm.at[pl.ds(sid * RPW, RPW)],
                    acc_sh.at[pl.ds(sid * RPW, RPW)])

    @pl.when(sid == 0)
    def _():
        pltpu.sync_copy(zvec_hbm, s_sh)

    plsc.subcore_barrier()

    mxa = mx_v[...]

    def idx_start(cc, uu):
        pltpu.async_copy(src_hbm.at[wid].at[cc], src_c.at[uu], sem_i[uu])
        pltpu.async_copy(dst_hbm.at[wid].at[cc], dst_c.at[uu], sem_i[uu])

    def idx_wait(cc, uu):
        pltpu.make_async_copy(src_hbm.at[wid].at[cc], src_c.at[uu],
                              sem_i[uu]).wait()
        pltpu.make_async_copy(dst_hbm.at[wid].at[cc], dst_c.at[uu],
                              sem_i[uu]).wait()

    GS = CHUNK // GSPLIT

    def gather_start(uu):
        for p in range(GSPLIT):
            pltpu.async_copy(
                xs_hbm.at[src_c.at[uu].at[pl.ds(p * GS, GS)]],
                rows_v.at[uu % 2].at[pl.ds(p * GS, GS)],
                sem_g[uu % 2])

    def gather_wait(uu):
        for p in range(GSPLIT):
            pltpu.make_async_copy(
                xs_hbm.at[src_c.at[uu].at[pl.ds(p * GS, GS)]],
                rows_v.at[uu % 2].at[pl.ds(p * GS, GS)],
                sem_g[uu % 2]).wait()

    def rows_scatter_start(uu):
        pltpu.async_copy(rows_f, acc_sh.at[dst_c.at[uu]], sem_sc, add=True)

    def rows_scatter_wait(uu):
        pltpu.make_async_copy(rows_f, acc_sh.at[dst_c.at[uu]],
                              sem_sc).wait()

    def s_scatter_start(uu):
        pltpu.async_copy(val_c.at[uu % 2], s_sh.at[dst_c.at[uu]],
                         sem_ss[uu % 2], add=True)

    def s_scatter_wait(uu):
        pltpu.make_async_copy(val_c.at[uu % 2], s_sh.at[dst_c.at[uu]],
                              sem_ss[uu % 2]).wait()

    def val_compute(uu):
        for i in range(CHUNK // 16):
            s_idx = src_c[uu, pl.ds(i * 16, 16)]
            d_idx = dst_c[uu, pl.ds(i * 16, 16)]
            a_s = plsc.load_gather(asad_v, [s_idx * 2])
            a_d = plsc.load_gather(asad_v, [d_idx * 2 + 1])
            t = a_s + a_d
            t = jnp.where(t >= 0.0, t, 0.2 * t)
            m = mxa + a_d
            m = jnp.where(m >= 0.0, m, 0.2 * m)
            val_c[uu % 2, pl.ds(i * 16, 16)] = jnp.exp(t - m)

    def scale_scatter(uu):
        gather_wait(uu)

        @pl.loop(0, CHUNK, unroll=2)
        def _scale(j):
            vb = plsc.load_gather(val_c.at[uu % 2],
                                  [jnp.full((16,), j, jnp.int32)])
            for k in range(4):
                w = rows_v[uu % 2, j, pl.ds(k * 16, 16)]
                ab = plsc.bitcast(w, jnp.bfloat16)
                a, b = plsc.unpack(ab, format=plsc.PackFormat.INTERLEAVED)
                rows_f[j, pl.ds(k * 32, 16)] = a * vb
                rows_f[j, pl.ds(k * 32 + 16, 16)] = b * vb

        rows_scatter_start(uu)

    idx_start(0, 0)
    idx_start(1, 1)

    @pl.loop(0, CPW // UNROLL)
    def _blk(g):
        for u in range(UNROLL):
            c = g * UNROLL + u
            um1 = (u - 1) % UNROLL
            up2 = (u + 2) % UNROLL

            def drains():
                rows_scatter_wait(up2)
                s_scatter_wait(up2)

            if u < 2:
                @pl.when(g > 0)
                def _():
                    drains()
            else:
                drains()
            idx_wait(c, u)
            gather_start(u)
            if u < 2:
                idx_start(c + 2, up2)
            else:
                @pl.when(g < CPW // UNROLL - 1)
                def _():
                    idx_start(c + 2, up2)
            val_compute(u)
            s_scatter_start(u)
            if u == 0:
                @pl.when(g > 0)
                def _():
                    scale_scatter(3)
            else:
                scale_scatter(u - 1)

    rows_scatter_wait(2)
    scale_scatter(3)
    rows_scatter_wait(3)
    s_scatter_wait(2)
    s_scatter_wait(3)

    plsc.subcore_barrier()

    pltpu.sync_copy(acc_sh.at[pl.ds(sid * RPW, RPW)],
                    acc_out.at[cid].at[pl.ds(sid * RPW, RPW)])

    @pl.when(sid == 0)
    def _():
        pltpu.sync_copy(s_sh, s_out.at[cid])


def _sc_edge(src3, dst3, asad_flat, mx, xs, zrows, zvec):
    mesh = plsc.VectorSubcoreMesh(core_axis_name="c", subcore_axis_name="s")
    f = pl.kernel(
        _sc_edge_body,
        out_type=[
            jax.ShapeDtypeStruct((2, NPAD, H), jnp.float32),
            jax.ShapeDtypeStruct((2, NPAD), jnp.float32),
        ],
        mesh=mesh,
        compiler_params=pltpu.CompilerParams(needs_layout_passes=False,
                                             use_tc_tiling_on_sc=False),
        scratch_types=[
            pltpu.VMEM((2 * NPAD,), jnp.float32),
            pltpu.VMEM((UNROLL, CHUNK), jnp.int32),
            pltpu.VMEM((UNROLL, CHUNK), jnp.int32),
            pltpu.VMEM((2, CHUNK), jnp.float32),
            pltpu.VMEM((2, CHUNK, H // 2), jnp.int32),
            pltpu.VMEM((CHUNK, H), jnp.float32),
            pltpu.VMEM((16,), jnp.float32),
        ] + [pltpu.SemaphoreType.DMA] * 9 + [
            pltpu.VMEM_SHARED((NPAD, H), jnp.float32),
            pltpu.VMEM_SHARED((NPAD,), jnp.float32),
        ],
    )
    return f(src3, dst3, asad_flat, mx, xs, zrows, zvec)


_PERM = []
for _k in range(H // 32):
    for _i in range(16):
        _PERM += [_k * 32 + _i, _k * 32 + 16 + _i]
_PERM = tuple(_PERM)


def kernel(x, edge_index, Ws1, Wd1, as1, ad1, b1, Wl1, bl1,
           Ws2, Wd2, as2, ad2, b2, Wl2, bl2):
    src = edge_index[0].astype(jnp.int32)
    dst = edge_index[1].astype(jnp.int32)
    pad = EPAD - E
    src3 = jnp.concatenate([src, jnp.full((pad,), N, jnp.int32)]
                           ).reshape(NW, CPW, CHUNK)
    dst3 = jnp.concatenate([dst, jnp.full((pad,), N, jnp.int32)]
                           ).reshape(NW, CPW, CHUNK)
    xp = jnp.pad(x, ((0, NPAD - N), (0, 0)))
    zrows = jnp.zeros((NPAD, H), jnp.float32)
    zvec = jnp.zeros((NPAD,), jnp.float32)
    perm = jnp.asarray(_PERM, jnp.int32)

    def packed(xs):
        return jax.lax.bitcast_convert_type(
            xs.reshape(NPAD, H // 2, 2), jnp.int32)

    xs1, asad1, mx1, lin1 = _prologue(xp, Ws1[:, perm], Wd1, as1[perm],
                                      ad1, Wl1, bl1)
    acc1, sv1 = _sc_edge(src3, dst3, asad1.reshape(2 * NPAD), mx1,
                         packed(xs1), zrows, zvec)
    xs2, asad2, mx2, lin2 = _mid(acc1, sv1, lin1, b1, Ws2[:, perm], Wd2,
                                 as2[perm], ad2, Wl2, bl2)
    acc2, sv2 = _sc_edge(src3, dst3, asad2.reshape(2 * NPAD), mx2,
                         packed(xs2), zrows, zvec)
    return _final(acc2, sv2, lin2, b2)[:N]
